# baseline (speedup 1.0000x reference)
import functools

import jax
import jax.numpy as jnp
from jax import lax
from jax.experimental import pallas as pl
from jax.experimental.pallas import tpu as pltpu
from jax.experimental.pallas import tpu_sc as plsc

_N = 10000
_E = 320000
_D = 128
_NC = 2
_NS = 16
_NW = _NC * _NS
_C = 40
_CHW = 250
_EPW = _C * _CHW
_NPAD = 10240
_RPT = _NPAD // _NS
_NBUF = 5
_NBK = _CHW // _NBUF


def _sc_agg_body(do_cnt, h_hbm, src_hbm, dst_hbm, agg_hbm, cnt_hbm,
                 src_blk, dst_blk, rows_v, ones_v, cz_v,
                 shared_agg, shared_cnt, sem_g, sem_s, sem_c, sem_is, sem_id):
    c = lax.axis_index("c")
    s = lax.axis_index("s")

    def _zb(t, _):
        i = t // (_D // 16)
        j = t - i * (_D // 16)
        rows_v[0][i, pl.ds(j * 16, 16)] = jnp.zeros((16,), jnp.float32)
        return 0
    lax.fori_loop(0, _C * (_D // 16), _zb, 0)
    base = s * _RPT
    for k in range(_RPT // _C):
        pltpu.async_copy(rows_v[0], shared_agg.at[pl.ds(base + k * _C, _C)],
                         sem_s[k % _NBUF])
    if do_cnt:
        for i in range(_RPT // 16):
            cz_v[pl.ds(i * 16, 16)] = jnp.zeros((16,), jnp.float32)
        pltpu.async_copy(cz_v, shared_cnt.at[pl.ds(base, _RPT)], sem_c[0])
        for i in range(3):
            ones_v[pl.ds(i * 16, 16)] = jnp.full((16,), 1.0, jnp.float32)
    for k in range(_RPT // _C):
        pltpu.make_async_copy(rows_v[0],
                              shared_agg.at[pl.ds(base + k * _C, _C)],
                              sem_s[k % _NBUF]).wait()
    if do_cnt:
        pltpu.make_async_copy(cz_v, shared_cnt.at[pl.ds(base, _RPT)],
                              sem_c[0]).wait()
    plsc.subcore_barrier()

    w0 = (c * _NS + s) * _EPW

    def _idx_load(g_blk, p, sync):
        srcs = src_hbm.at[pl.ds(w0 + g_blk * _NBUF * _C, _NBUF * _C)]
        dsts = dst_hbm.at[pl.ds(w0 + g_blk * _NBUF * _C, _NBUF * _C)]
        if sync:
            pltpu.sync_copy(srcs, src_blk[p])
            pltpu.sync_copy(dsts, dst_blk[p])
        else:
            pltpu.async_copy(srcs, src_blk[p], sem_is[p])
            pltpu.async_copy(dsts, dst_blk[p], sem_id[p])

    def _idx_wait(g_blk, p):
        pltpu.make_async_copy(
            src_hbm.at[pl.ds(w0 + g_blk * _NBUF * _C, _NBUF * _C)],
            src_blk[p], sem_is[p]).wait()
        pltpu.make_async_copy(
            dst_hbm.at[pl.ds(w0 + g_blk * _NBUF * _C, _NBUF * _C)],
            dst_blk[p], sem_id[p]).wait()

    _idx_load(0, 0, True)
    _idx_load(1, 1, False)
    for b in range(_NBUF):
        pltpu.async_copy(h_hbm.at[src_blk[0].at[pl.ds(b * _C, _C)]], rows_v[b], sem_g[b])

    def _super(gg, _):
        for q in range(2):
            g = 2 * gg + q
            p = q
            for b in range(_NBUF):
                pltpu.make_async_copy(h_hbm.at[src_blk[p].at[pl.ds(b * _C, _C)]],
                                      rows_v[b], sem_g[b]).wait()
                pltpu.async_copy(rows_v[b], shared_agg.at[dst_blk[p].at[pl.ds(b * _C, _C)]],
                                 sem_s[b], add=True)
                if do_cnt:
                    pltpu.async_copy(ones_v.at[pl.ds(0, _C)],
                                     shared_cnt.at[dst_blk[p].at[pl.ds(b * _C, _C)]],
                                     sem_c[b], add=True)

            @pl.when(g < _NBK - 1)
            def _():
                _idx_wait(g + 1, 1 - p)
            for b in range(_NBUF):
                pltpu.make_async_copy(rows_v[b],
                                      shared_agg.at[dst_blk[p].at[pl.ds(b * _C, _C)]],
                                      sem_s[b]).wait()
                if do_cnt:
                    pltpu.make_async_copy(ones_v.at[pl.ds(0, _C)],
                                          shared_cnt.at[dst_blk[p].at[pl.ds(b * _C, _C)]],
                                          sem_c[b]).wait()

                @pl.when(g < _NBK - 1)
                def _():
                    pltpu.async_copy(h_hbm.at[src_blk[1 - p].at[pl.ds(b * _C, _C)]],
                                     rows_v[b], sem_g[b])

            @pl.when(g < _NBK - 2)
            def _():
                _idx_load(g + 2, p, False)
        return 0
    lax.fori_loop(0, _NBK // 2, _super, 0)

    plsc.subcore_barrier()
    pltpu.sync_copy(shared_agg.at[pl.ds(base, _RPT)],
                    agg_hbm.at[c, pl.ds(base, _RPT)])
    if do_cnt:
        pltpu.sync_copy(shared_cnt.at[pl.ds(base, _RPT)],
                        cnt_hbm.at[c, pl.ds(base, _RPT)])


def _make_sc_agg(do_cnt):
    mesh = plsc.VectorSubcoreMesh(core_axis_name="c", subcore_axis_name="s")
    out_type = [jax.ShapeDtypeStruct((_NC, _NPAD, _D), jnp.float32),
                jax.ShapeDtypeStruct((_NC, _NPAD), jnp.float32)]
    return pl.kernel(
        functools.partial(_sc_agg_body, do_cnt),
        out_type=out_type,
        mesh=mesh,
        scratch_types=[
            [pltpu.VMEM((_NBUF * _C,), jnp.int32) for _ in range(2)],
            [pltpu.VMEM((_NBUF * _C,), jnp.int32) for _ in range(2)],
            [pltpu.VMEM((_C, _D), jnp.float32) for _ in range(_NBUF)],
            pltpu.VMEM((48,), jnp.float32),
            pltpu.VMEM((_RPT,), jnp.float32),
            pltpu.VMEM_SHARED((_NPAD, _D), jnp.float32),
            pltpu.VMEM_SHARED((_NPAD,), jnp.float32),
            [pltpu.SemaphoreType.DMA for _ in range(_NBUF)],
            [pltpu.SemaphoreType.DMA for _ in range(_NBUF)],
            [pltpu.SemaphoreType.DMA for _ in range(_NBUF)],
            [pltpu.SemaphoreType.DMA for _ in range(2)],
            [pltpu.SemaphoreType.DMA for _ in range(2)],
        ],
    )


_sc_agg_cnt = _make_sc_agg(True)
_sc_agg_nocnt = _make_sc_agg(False)

_BLK = 10000


def _dense_body(h_ref, p_ref, cnt_ref, wl_ref, wr_ref, b_ref, o_ref):
    cnt = cnt_ref[:, 0] + cnt_ref[:, 1]
    agg = p_ref[0] + p_ref[1]
    mean = agg / jnp.maximum(cnt, 1.0)[:, None]
    out = (lax.dot_general(mean, wl_ref[...], (((1,), (1,)), ((), ())),
                           preferred_element_type=jnp.float32)
           + lax.dot_general(h_ref[...], wr_ref[...], (((1,), (1,)), ((), ())),
                             preferred_element_type=jnp.float32)
           + b_ref[...])
    o_ref[...] = jnp.maximum(out, 0.0)


_dense = pl.pallas_call(
    _dense_body,
    grid=(_N // _BLK,),
    in_specs=[
        pl.BlockSpec((_BLK, _D), lambda i: (i, 0)),
        pl.BlockSpec((_NC, _BLK, _D), lambda i: (0, i, 0)),
        pl.BlockSpec((_BLK, _NC), lambda i: (i, 0)),
        pl.BlockSpec((_D, _D), lambda i: (0, 0)),
        pl.BlockSpec((_D, _D), lambda i: (0, 0)),
        pl.BlockSpec((1, _D), lambda i: (0, 0)),
    ],
    out_specs=pl.BlockSpec((_BLK, _D), lambda i: (i, 0)),
    out_shape=jax.ShapeDtypeStruct((_N, _D), jnp.float32),
)


def kernel(x, edge_index, W1l, W1r, b1, W2l, W2r, b2):
    x = x.astype(jnp.float32)
    src = edge_index[0]
    dst = edge_index[1]
    aggp, cntp = _sc_agg_cnt(x, src, dst)
    cnt_t = cntp.T
    h1 = _dense(x, aggp, cnt_t, W1l, W1r, b1.reshape(1, _D))
    aggp2, _ = _sc_agg_nocnt(h1, src, dst)
    out = _dense(h1, aggp2, cnt_t, W2l, W2r, b2.reshape(1, _D))
    return out


# speedup vs baseline: 1.0221x; 1.0221x over previous
import functools

import jax
import jax.numpy as jnp
from jax import lax
from jax.experimental import pallas as pl
from jax.experimental.pallas import tpu as pltpu
from jax.experimental.pallas import tpu_sc as plsc

_N = 10000
_E = 320000
_D = 128
_NC = 2
_NS = 16
_NW = _NC * _NS
_C = 40
_CHW = 250
_EPW = _C * _CHW
_NPAD = 10240
_RPT = _NPAD // _NS
_NBUF = 5
_NBK = _CHW // _NBUF


def _sc_agg_body(do_cnt, h_hbm, src_hbm, dst_hbm, agg_hbm, cnt_hbm,
                 src_blk, dst_blk, rows_v, ones_v, cz_v,
                 shared_agg, shared_cnt, sem_g, sem_s, sem_c, sem_is, sem_id):
    c = lax.axis_index("c")
    s = lax.axis_index("s")

    w0 = (c * _NS + s) * _EPW

    def _idx_load(g_blk, p, sync):
        srcs = src_hbm.at[pl.ds(w0 + g_blk * _NBUF * _C, _NBUF * _C)]
        dsts = dst_hbm.at[pl.ds(w0 + g_blk * _NBUF * _C, _NBUF * _C)]
        if sync:
            pltpu.sync_copy(srcs, src_blk[p])
            pltpu.sync_copy(dsts, dst_blk[p])
        else:
            pltpu.async_copy(srcs, src_blk[p], sem_is[p])
            pltpu.async_copy(dsts, dst_blk[p], sem_id[p])

    def _idx_wait(g_blk, p):
        pltpu.make_async_copy(
            src_hbm.at[pl.ds(w0 + g_blk * _NBUF * _C, _NBUF * _C)],
            src_blk[p], sem_is[p]).wait()
        pltpu.make_async_copy(
            dst_hbm.at[pl.ds(w0 + g_blk * _NBUF * _C, _NBUF * _C)],
            dst_blk[p], sem_id[p]).wait()

    _idx_load(0, 0, False)
    _idx_load(1, 1, False)

    def _zb(t, _):
        i = t // (_D // 16)
        j = t - i * (_D // 16)
        rows_v[0][i, pl.ds(j * 16, 16)] = jnp.zeros((16,), jnp.float32)
        return 0
    lax.fori_loop(0, _C * (_D // 16), _zb, 0)
    base = s * _RPT
    for k in range(_RPT // _C):
        pltpu.async_copy(rows_v[0], shared_agg.at[pl.ds(base + k * _C, _C)],
                         sem_s[k % _NBUF])
    if do_cnt:
        for i in range(_RPT // 16):
            cz_v[pl.ds(i * 16, 16)] = jnp.zeros((16,), jnp.float32)
        pltpu.async_copy(cz_v, shared_cnt.at[pl.ds(base, _RPT)], sem_c[0])
        for i in range(3):
            ones_v[pl.ds(i * 16, 16)] = jnp.full((16,), 1.0, jnp.float32)
    for k in range(_RPT // _C):
        pltpu.make_async_copy(rows_v[0],
                              shared_agg.at[pl.ds(base + k * _C, _C)],
                              sem_s[k % _NBUF]).wait()
    if do_cnt:
        pltpu.make_async_copy(cz_v, shared_cnt.at[pl.ds(base, _RPT)],
                              sem_c[0]).wait()
    _idx_wait(0, 0)
    for b in range(_NBUF):
        pltpu.async_copy(h_hbm.at[src_blk[0].at[pl.ds(b * _C, _C)]], rows_v[b], sem_g[b])
    plsc.subcore_barrier()

    def _super(gg, _):
        for q in range(2):
            g = 2 * gg + q
            p = q
            for b in range(_NBUF):
                pltpu.make_async_copy(h_hbm.at[src_blk[p].at[pl.ds(b * _C, _C)]],
                                      rows_v[b], sem_g[b]).wait()
                pltpu.async_copy(rows_v[b], shared_agg.at[dst_blk[p].at[pl.ds(b * _C, _C)]],
                                 sem_s[b], add=True)
                if do_cnt:
                    pltpu.async_copy(ones_v.at[pl.ds(0, _C)],
                                     shared_cnt.at[dst_blk[p].at[pl.ds(b * _C, _C)]],
                                     sem_c[b], add=True)

            @pl.when(g < _NBK - 1)
            def _():
                _idx_wait(g + 1, 1 - p)
            for b in range(_NBUF):
                pltpu.make_async_copy(rows_v[b],
                                      shared_agg.at[dst_blk[p].at[pl.ds(b * _C, _C)]],
                                      sem_s[b]).wait()
                if do_cnt:
                    pltpu.make_async_copy(ones_v.at[pl.ds(0, _C)],
                                          shared_cnt.at[dst_blk[p].at[pl.ds(b * _C, _C)]],
                                          sem_c[b]).wait()

                @pl.when(g < _NBK - 1)
                def _():
                    pltpu.async_copy(h_hbm.at[src_blk[1 - p].at[pl.ds(b * _C, _C)]],
                                     rows_v[b], sem_g[b])

            @pl.when(g < _NBK - 2)
            def _():
                _idx_load(g + 2, p, False)
        return 0
    lax.fori_loop(0, _NBK // 2, _super, 0)

    plsc.subcore_barrier()
    pltpu.sync_copy(shared_agg.at[pl.ds(base, _RPT)],
                    agg_hbm.at[c, pl.ds(base, _RPT)])
    if do_cnt:
        pltpu.sync_copy(shared_cnt.at[pl.ds(base, _RPT)],
                        cnt_hbm.at[c, pl.ds(base, _RPT)])


def _make_sc_agg(do_cnt):
    mesh = plsc.VectorSubcoreMesh(core_axis_name="c", subcore_axis_name="s")
    out_type = [jax.ShapeDtypeStruct((_NC, _NPAD, _D), jnp.float32),
                jax.ShapeDtypeStruct((_NC, _NPAD), jnp.float32)]
    return pl.kernel(
        functools.partial(_sc_agg_body, do_cnt),
        out_type=out_type,
        mesh=mesh,
        scratch_types=[
            [pltpu.VMEM((_NBUF * _C,), jnp.int32) for _ in range(2)],
            [pltpu.VMEM((_NBUF * _C,), jnp.int32) for _ in range(2)],
            [pltpu.VMEM((_C, _D), jnp.float32) for _ in range(_NBUF)],
            pltpu.VMEM((48,), jnp.float32),
            pltpu.VMEM((_RPT,), jnp.float32),
            pltpu.VMEM_SHARED((_NPAD, _D), jnp.float32),
            pltpu.VMEM_SHARED((_NPAD,), jnp.float32),
            [pltpu.SemaphoreType.DMA for _ in range(_NBUF)],
            [pltpu.SemaphoreType.DMA for _ in range(_NBUF)],
            [pltpu.SemaphoreType.DMA for _ in range(_NBUF)],
            [pltpu.SemaphoreType.DMA for _ in range(2)],
            [pltpu.SemaphoreType.DMA for _ in range(2)],
        ],
    )


_sc_agg_cnt = _make_sc_agg(True)
_sc_agg_nocnt = _make_sc_agg(False)

_BLK = 5000


def _dense_body(h_ref, p_ref, cnt_ref, wl_ref, wr_ref, b_ref, o_ref):
    cnt = cnt_ref[:, 0] + cnt_ref[:, 1]
    agg = p_ref[0] + p_ref[1]
    mean = agg / jnp.maximum(cnt, 1.0)[:, None]
    out = (lax.dot_general(mean, wl_ref[...], (((1,), (1,)), ((), ())),
                           preferred_element_type=jnp.float32)
           + lax.dot_general(h_ref[...], wr_ref[...], (((1,), (1,)), ((), ())),
                             preferred_element_type=jnp.float32)
           + b_ref[...])
    o_ref[...] = jnp.maximum(out, 0.0)


_dense = pl.pallas_call(
    _dense_body,
    grid=(_N // _BLK,),
    in_specs=[
        pl.BlockSpec((_BLK, _D), lambda i: (i, 0)),
        pl.BlockSpec((_NC, _BLK, _D), lambda i: (0, i, 0)),
        pl.BlockSpec((_BLK, _NC), lambda i: (i, 0)),
        pl.BlockSpec((_D, _D), lambda i: (0, 0)),
        pl.BlockSpec((_D, _D), lambda i: (0, 0)),
        pl.BlockSpec((1, _D), lambda i: (0, 0)),
    ],
    out_specs=pl.BlockSpec((_BLK, _D), lambda i: (i, 0)),
    out_shape=jax.ShapeDtypeStruct((_N, _D), jnp.float32),
)


def kernel(x, edge_index, W1l, W1r, b1, W2l, W2r, b2):
    x = x.astype(jnp.float32)
    src = edge_index[0]
    dst = edge_index[1]
    aggp, cntp = _sc_agg_cnt(x, src, dst)
    cnt_t = cntp.T
    h1 = _dense(x, aggp, cnt_t, W1l, W1r, b1.reshape(1, _D))
    aggp2, _ = _sc_agg_nocnt(h1, src, dst)
    out = _dense(h1, aggp2, cnt_t, W2l, W2r, b2.reshape(1, _D))
    return out
